# baseline (speedup 1.0000x reference)
import functools

import jax
import jax.numpy as jnp
from jax.experimental import pallas as pl

N_EXPERTS = 8
TOP_K = 2
BLOCK_T = 2048


def _gate_kernel(x_ref, w_ref, probs_ref, tv_ref, ti_ref):
    x = x_ref[...]
    w = w_ref[...]
    scores = jax.lax.dot_general(
        x, w, (((1,), (1,)), ((), ())), preferred_element_type=jnp.float32
    )
    m = jnp.max(scores, axis=-1, keepdims=True)
    e = jnp.exp(scores - m)
    s = jnp.sum(e, axis=-1, keepdims=True)
    probs = e / s
    probs_ref[...] = probs

    v1 = jnp.max(probs, axis=-1)
    i1 = jnp.argmax(probs, axis=-1)
    lane = jax.lax.broadcasted_iota(jnp.int32, probs.shape, 1)
    masked = jnp.where(lane == i1[:, None], -jnp.inf, probs)
    v2 = jnp.max(masked, axis=-1)
    i2 = jnp.argmax(masked, axis=-1)
    denom = v1 + v2 + 1e-9
    tv_ref[...] = jnp.stack([v1 / denom, v2 / denom], axis=-1)
    ti_ref[...] = jnp.stack([i1, i2], axis=-1).astype(jnp.int32)


@functools.partial(jax.jit, static_argnames=())
def kernel(x, weight):
    n_tok, dim = x.shape
    n_exp = weight.shape[0]
    grid = (n_tok // BLOCK_T,)
    probs, tv, ti = pl.pallas_call(
        _gate_kernel,
        grid=grid,
        in_specs=[
            pl.BlockSpec((BLOCK_T, dim), lambda i: (i, 0)),
            pl.BlockSpec((n_exp, dim), lambda i: (0, 0)),
        ],
        out_specs=[
            pl.BlockSpec((BLOCK_T, n_exp), lambda i: (i, 0)),
            pl.BlockSpec((BLOCK_T, TOP_K), lambda i: (i, 0)),
            pl.BlockSpec((BLOCK_T, TOP_K), lambda i: (i, 0)),
        ],
        out_shape=[
            jax.ShapeDtypeStruct((n_tok, n_exp), jnp.float32),
            jax.ShapeDtypeStruct((n_tok, TOP_K), jnp.float32),
            jax.ShapeDtypeStruct((n_tok, TOP_K), jnp.int32),
        ],
    )(x, weight)
    return tv, ti, probs


# speedup vs baseline: 2.3284x; 2.3284x over previous
import jax
import jax.numpy as jnp
from jax.experimental import pallas as pl

N_EXPERTS = 8
TOP_K = 2
BLOCK_T = 2048


def _gate_kernel(x_ref, w_ref, probs_ref, tv_ref, ti_ref):
    x = x_ref[...]
    w = w_ref[...]
    scores = jax.lax.dot_general(
        w, x, (((1,), (1,)), ((), ())), preferred_element_type=jnp.float32
    )
    m = jnp.max(scores, axis=0, keepdims=True)
    e = jnp.exp(scores - m)
    s = jnp.sum(e, axis=0, keepdims=True)
    probs = e / s
    probs_ref[...] = probs

    v1 = jnp.max(probs, axis=0, keepdims=True)
    i1 = jnp.argmax(probs, axis=0).reshape(1, -1)
    row = jax.lax.broadcasted_iota(jnp.int32, probs.shape, 0)
    masked = jnp.where(row == i1, -jnp.inf, probs)
    v2 = jnp.max(masked, axis=0, keepdims=True)
    i2 = jnp.argmax(masked, axis=0).reshape(1, -1)
    denom = v1 + v2 + 1e-9
    tv_ref[...] = jnp.concatenate([v1 / denom, v2 / denom], axis=0)
    ti_ref[...] = jnp.concatenate([i1, i2], axis=0).astype(jnp.int32)


def kernel(x, weight):
    n_tok, dim = x.shape
    n_exp = weight.shape[0]
    grid = (n_tok // BLOCK_T,)
    probs_t, tv_t, ti_t = pl.pallas_call(
        _gate_kernel,
        grid=grid,
        in_specs=[
            pl.BlockSpec((BLOCK_T, dim), lambda i: (i, 0)),
            pl.BlockSpec((n_exp, dim), lambda i: (0, 0)),
        ],
        out_specs=[
            pl.BlockSpec((n_exp, BLOCK_T), lambda i: (0, i)),
            pl.BlockSpec((TOP_K, BLOCK_T), lambda i: (0, i)),
            pl.BlockSpec((TOP_K, BLOCK_T), lambda i: (0, i)),
        ],
        out_shape=[
            jax.ShapeDtypeStruct((n_exp, n_tok), jnp.float32),
            jax.ShapeDtypeStruct((TOP_K, n_tok), jnp.float32),
            jax.ShapeDtypeStruct((TOP_K, n_tok), jnp.int32),
        ],
    )(x, weight)
    return tv_t.T, ti_t.T, probs_t.T


# speedup vs baseline: 2.3856x; 1.0246x over previous
import jax
import jax.numpy as jnp
from jax.experimental import pallas as pl

N_EXPERTS = 8
TOP_K = 2
BLOCK_T = 4096


def _gate_kernel(x_ref, w_ref, probs_ref, tv_ref, ti_ref):
    x = x_ref[...]
    w = w_ref[...]
    scores = jax.lax.dot_general(
        w, x, (((1,), (1,)), ((), ())), preferred_element_type=jnp.float32
    )
    m = jnp.max(scores, axis=0, keepdims=True)
    e = jnp.exp(scores - m)
    s = jnp.sum(e, axis=0, keepdims=True)
    probs = e / s
    probs_ref[...] = probs

    v1 = jnp.max(probs, axis=0, keepdims=True)
    i1 = jnp.argmax(probs, axis=0).reshape(1, -1)
    row = jax.lax.broadcasted_iota(jnp.int32, probs.shape, 0)
    masked = jnp.where(row == i1, -jnp.inf, probs)
    v2 = jnp.max(masked, axis=0, keepdims=True)
    i2 = jnp.argmax(masked, axis=0).reshape(1, -1)
    denom = v1 + v2 + 1e-9
    tv_ref[...] = jnp.concatenate([v1 / denom, v2 / denom], axis=0)
    ti_ref[...] = jnp.concatenate([i1, i2], axis=0).astype(jnp.int32)


def kernel(x, weight):
    n_tok, dim = x.shape
    n_exp = weight.shape[0]
    grid = (n_tok // BLOCK_T,)
    probs_t, tv_t, ti_t = pl.pallas_call(
        _gate_kernel,
        grid=grid,
        in_specs=[
            pl.BlockSpec((BLOCK_T, dim), lambda i: (i, 0)),
            pl.BlockSpec((n_exp, dim), lambda i: (0, 0)),
        ],
        out_specs=[
            pl.BlockSpec((n_exp, BLOCK_T), lambda i: (0, i)),
            pl.BlockSpec((TOP_K, BLOCK_T), lambda i: (0, i)),
            pl.BlockSpec((TOP_K, BLOCK_T), lambda i: (0, i)),
        ],
        out_shape=[
            jax.ShapeDtypeStruct((n_exp, n_tok), jnp.float32),
            jax.ShapeDtypeStruct((TOP_K, n_tok), jnp.float32),
            jax.ShapeDtypeStruct((TOP_K, n_tok), jnp.int32),
        ],
    )(x, weight)
    return tv_t.T, ti_t.T, probs_t.T
